# baseline (speedup 1.0000x reference)
import functools

import jax
import jax.numpy as jnp
from jax import lax
from jax.experimental import pallas as pl
from jax.experimental.pallas import tpu as pltpu
from jax.experimental.pallas import tpu_sc as plsc

_N, _D, _H, _E = 32768, 768, 128, 64


_BN = 1024


def _mlp_body(x_ref, w1_ref, b1_ref, w2_ref, b2_ref, out_ref):
    h = jnp.dot(x_ref[...], w1_ref[...], preferred_element_type=jnp.float32)
    h = h + b1_ref[...]
    h = h * jax.nn.sigmoid(h)
    out_ref[...] = (
        jnp.dot(h, w2_ref[...], preferred_element_type=jnp.float32) + b2_ref[...]
    )


def _mlp_logits(x, w1, b1, w2, b2, row0, nrows):
    blk0 = row0 // _BN
    return pl.pallas_call(
        _mlp_body,
        grid=(nrows // _BN,),
        in_specs=[
            pl.BlockSpec((_BN, _D), lambda i: (blk0 + i, 0)),
            pl.BlockSpec((_D, _H), lambda i: (0, 0)),
            pl.BlockSpec((1, _H), lambda i: (0, 0)),
            pl.BlockSpec((_H, _E), lambda i: (0, 0)),
            pl.BlockSpec((1, _E), lambda i: (0, 0)),
        ],
        out_specs=pl.BlockSpec((_BN, _E), lambda i: (i, 0)),
        out_shape=jax.ShapeDtypeStruct((nrows, _E), jnp.float32),
    )(x, w1, b1.reshape(1, _H), w2, b2.reshape(1, _E))


_NC, _NS, _L = 2, 16, 16
_NW = _NC * _NS
_RPW = _N // _NW
_CH = 128
_G = _CH // _L
_NCHUNK = _RPW // _CH
_IL = 4


def _topk_body(rpw, logits_hbm, out_hbm, log_v, out_v, stash_v, in_sem, out_sem):
    wid = lax.axis_index("s") * _NC + lax.axis_index("c")
    lanes = lax.iota(jnp.int32, _L)
    zeros = jnp.zeros((_L,), jnp.float32)
    neg_inf = jnp.full((_L,), -jnp.inf, jnp.float32)
    izero = jnp.zeros((_L,), jnp.int32)
    nchunk = rpw // _CH

    def _roff(ci):
        return wid * rpw + ci * _CH

    def _zero(i, _):
        out_v[0][i // 4, pl.ds((i % 4) * _L, _L)] = zeros
        out_v[1][i // 4, pl.ds((i % 4) * _L, _L)] = zeros
        return 0

    lax.fori_loop(0, (_CH * _E) // _L, _zero, 0)

    def _compute(p, stash):
        @plsc.parallel_loop(0, _G, unroll=_IL)
        def _group(g):
            rl = g * _L + lanes
            m1, m2, i1, i2 = neg_inf, neg_inf, izero, izero
            for e in range(_E):
                ev = jnp.full((_L,), e, jnp.int32)
                v = plsc.load_gather(log_v[p], [rl, ev])
                gt1 = v > m1
                gt2 = v > m2
                m2n = jnp.where(gt1, m1, jnp.where(gt2, v, m2))
                i2n = jnp.where(gt1, i1, jnp.where(gt2, ev, i2))
                m1 = jnp.where(gt1, v, m1)
                i1 = jnp.where(gt1, ev, i1)
                m2, i2 = m2n, i2n
            t = jnp.exp(m2 - m1)
            denom = 1.0 + t
            plsc.store_scatter(out_v[p], [rl, i1], 1.0 / denom)
            plsc.store_scatter(out_v[p], [rl, i2], t / denom)
            if stash:
                stash_v[pl.ds((p * _G + g) * 2 * _L, _L)] = i1
                stash_v[pl.ds((p * _G + g) * 2 * _L + _L, _L)] = i2

    def _unset_zeros(p):
        def _unset(g, _):
            rl = g * _L + lanes
            s0 = pl.ds((p * _G + g) * 2 * _L, _L)
            s1 = pl.ds((p * _G + g) * 2 * _L + _L, _L)
            plsc.store_scatter(out_v[p], [rl, stash_v[s0]], zeros)
            plsc.store_scatter(out_v[p], [rl, stash_v[s1]], zeros)
            return 0

        lax.fori_loop(0, _G, _unset, 0)

    for ci in range(min(2, nchunk)):
        pltpu.async_copy(
            logits_hbm.at[pl.ds(_roff(ci), _CH)], log_v[ci % 2], in_sem[ci % 2]
        )
    for ci in range(nchunk):
        p = ci % 2
        pltpu.make_async_copy(
            logits_hbm.at[pl.ds(_roff(ci), _CH)], log_v[p], in_sem[p]
        ).wait()
        if ci >= 2:
            pltpu.make_async_copy(
                out_v[p], out_hbm.at[pl.ds(_roff(ci - 2), _CH)], out_sem[p]
            ).wait()
            _unset_zeros(p)
        _compute(p, stash=ci + 2 < nchunk)
        pltpu.async_copy(out_v[p], out_hbm.at[pl.ds(_roff(ci), _CH)], out_sem[p])
        if ci + 2 < nchunk:
            pltpu.async_copy(
                logits_hbm.at[pl.ds(_roff(ci + 2), _CH)], log_v[p], in_sem[p]
            )
    for ci in range(max(0, nchunk - 2), nchunk):
        p = ci % 2
        pltpu.make_async_copy(
            out_v[p], out_hbm.at[pl.ds(_roff(ci), _CH)], out_sem[p]
        ).wait()


@functools.lru_cache(maxsize=None)
def _make_sc_topk(nrows):
    rpw = nrows // _NW

    @functools.partial(
        pl.kernel,
        out_type=jax.ShapeDtypeStruct((nrows, _E), jnp.float32),
        mesh=plsc.VectorSubcoreMesh(
            core_axis_name="c", subcore_axis_name="s", num_cores=_NC, num_subcores=_NS
        ),
        scratch_types=[
            pltpu.VMEM((_CH, _E), jnp.float32),
            pltpu.VMEM((_CH, _E), jnp.float32),
            pltpu.VMEM((_CH, _E), jnp.float32),
            pltpu.VMEM((_CH, _E), jnp.float32),
            pltpu.VMEM((2 * _G * 2 * _L,), jnp.int32),
            pltpu.SemaphoreType.DMA,
            pltpu.SemaphoreType.DMA,
            pltpu.SemaphoreType.DMA,
            pltpu.SemaphoreType.DMA,
        ],
        compiler_params=pltpu.CompilerParams(needs_layout_passes=False),
    )
    def _sc_topk(logits_hbm, out_hbm, lv0, lv1, ov0, ov1, stash_v, is0, is1, os0, os1):
        _topk_body(
            rpw, logits_hbm, out_hbm, (lv0, lv1), (ov0, ov1), stash_v, (is0, is1), (os0, os1)
        )

    return _sc_topk


_NSLICE = 8


def kernel(global_features, W1, b1, W2, b2):
    nrows = _N // _NSLICE
    sc_topk = _make_sc_topk(nrows)
    outs = []
    for s in range(_NSLICE):
        logits = _mlp_logits(global_features, W1, b1, W2, b2, s * nrows, nrows)
        outs.append(sc_topk(logits))
    return jnp.concatenate(outs, axis=0)


# speedup vs baseline: 1.1980x; 1.1980x over previous
import functools

import jax
import jax.numpy as jnp
from jax import lax
from jax.experimental import pallas as pl
from jax.experimental.pallas import tpu as pltpu
from jax.experimental.pallas import tpu_sc as plsc

_N, _D, _H, _E = 32768, 768, 128, 64


_BN = 1024


def _mlp_body(x_ref, w1_ref, b1_ref, w2_ref, b2_ref, out_ref):
    h = jnp.dot(x_ref[...], w1_ref[...], preferred_element_type=jnp.float32)
    h = h + b1_ref[...]
    h = h * jax.nn.sigmoid(h)
    out_ref[...] = (
        jnp.dot(h, w2_ref[...], preferred_element_type=jnp.float32) + b2_ref[...]
    )


def _mlp_logits(x, w1, b1, w2, b2, row0, nrows):
    blk0 = row0 // _BN
    return pl.pallas_call(
        _mlp_body,
        grid=(nrows // _BN,),
        in_specs=[
            pl.BlockSpec((_BN, _D), lambda i: (blk0 + i, 0)),
            pl.BlockSpec((_D, _H), lambda i: (0, 0)),
            pl.BlockSpec((1, _H), lambda i: (0, 0)),
            pl.BlockSpec((_H, _E), lambda i: (0, 0)),
            pl.BlockSpec((1, _E), lambda i: (0, 0)),
        ],
        out_specs=pl.BlockSpec((_BN, _E), lambda i: (i, 0)),
        out_shape=jax.ShapeDtypeStruct((nrows, _E), jnp.float32),
    )(x, w1, b1.reshape(1, _H), w2, b2.reshape(1, _E))


_NC, _NS, _L = 2, 16, 16
_NW = _NC * _NS
_RPW = _N // _NW
_CH = 128
_G = _CH // _L
_NCHUNK = _RPW // _CH
_IL = 4


def _topk_body(rpw, logits_hbm, out_hbm, log_v, out_v, stash_v, in_sem, out_sem):
    wid = lax.axis_index("s") * _NC + lax.axis_index("c")
    lanes = lax.iota(jnp.int32, _L)
    zeros = jnp.zeros((_L,), jnp.float32)
    neg_inf = jnp.full((_L,), -jnp.inf, jnp.float32)
    izero = jnp.zeros((_L,), jnp.int32)
    nchunk = rpw // _CH

    def _roff(ci):
        return wid * rpw + ci * _CH

    def _zero(i, _):
        out_v[0][i // 4, pl.ds((i % 4) * _L, _L)] = zeros
        out_v[1][i // 4, pl.ds((i % 4) * _L, _L)] = zeros
        return 0

    lax.fori_loop(0, (_CH * _E) // _L, _zero, 0)

    def _compute(p, stash):
        @plsc.parallel_loop(0, _G, unroll=_IL)
        def _group(g):
            rl = g * _L + lanes
            m1, m2, i1, i2 = neg_inf, neg_inf, izero, izero
            for e in range(_E):
                ev = jnp.full((_L,), e, jnp.int32)
                v = plsc.load_gather(log_v[p], [rl, ev])
                gt1 = v > m1
                gt2 = v > m2
                m2n = jnp.where(gt1, m1, jnp.where(gt2, v, m2))
                i2n = jnp.where(gt1, i1, jnp.where(gt2, ev, i2))
                m1 = jnp.where(gt1, v, m1)
                i1 = jnp.where(gt1, ev, i1)
                m2, i2 = m2n, i2n
            t = jnp.exp(m2 - m1)
            denom = 1.0 + t
            plsc.store_scatter(out_v[p], [rl, i1], 1.0 / denom)
            plsc.store_scatter(out_v[p], [rl, i2], t / denom)
            if stash:
                stash_v[pl.ds((p * _G + g) * 2 * _L, _L)] = i1
                stash_v[pl.ds((p * _G + g) * 2 * _L + _L, _L)] = i2

    def _unset_zeros(p):
        def _unset(g, _):
            rl = g * _L + lanes
            s0 = pl.ds((p * _G + g) * 2 * _L, _L)
            s1 = pl.ds((p * _G + g) * 2 * _L + _L, _L)
            plsc.store_scatter(out_v[p], [rl, stash_v[s0]], zeros)
            plsc.store_scatter(out_v[p], [rl, stash_v[s1]], zeros)
            return 0

        lax.fori_loop(0, _G, _unset, 0)

    for ci in range(min(2, nchunk)):
        pltpu.async_copy(
            logits_hbm.at[pl.ds(_roff(ci), _CH)], log_v[ci % 2], in_sem[ci % 2]
        )
    for ci in range(nchunk):
        p = ci % 2
        pltpu.make_async_copy(
            logits_hbm.at[pl.ds(_roff(ci), _CH)], log_v[p], in_sem[p]
        ).wait()
        if ci >= 2:
            pltpu.make_async_copy(
                out_v[p], out_hbm.at[pl.ds(_roff(ci - 2), _CH)], out_sem[p]
            ).wait()
            _unset_zeros(p)
        _compute(p, stash=ci + 2 < nchunk)
        pltpu.async_copy(out_v[p], out_hbm.at[pl.ds(_roff(ci), _CH)], out_sem[p])
        if ci + 2 < nchunk:
            pltpu.async_copy(
                logits_hbm.at[pl.ds(_roff(ci + 2), _CH)], log_v[p], in_sem[p]
            )
    for ci in range(max(0, nchunk - 2), nchunk):
        p = ci % 2
        pltpu.make_async_copy(
            out_v[p], out_hbm.at[pl.ds(_roff(ci), _CH)], out_sem[p]
        ).wait()


@functools.lru_cache(maxsize=None)
def _make_sc_topk(nrows):
    rpw = nrows // _NW

    @functools.partial(
        pl.kernel,
        out_type=jax.ShapeDtypeStruct((nrows, _E), jnp.float32),
        mesh=plsc.VectorSubcoreMesh(
            core_axis_name="c", subcore_axis_name="s", num_cores=_NC, num_subcores=_NS
        ),
        scratch_types=[
            pltpu.VMEM((_CH, _E), jnp.float32),
            pltpu.VMEM((_CH, _E), jnp.float32),
            pltpu.VMEM((_CH, _E), jnp.float32),
            pltpu.VMEM((_CH, _E), jnp.float32),
            pltpu.VMEM((2 * _G * 2 * _L,), jnp.int32),
            pltpu.SemaphoreType.DMA,
            pltpu.SemaphoreType.DMA,
            pltpu.SemaphoreType.DMA,
            pltpu.SemaphoreType.DMA,
        ],
        compiler_params=pltpu.CompilerParams(needs_layout_passes=False),
    )
    def _sc_topk(logits_hbm, out_hbm, lv0, lv1, ov0, ov1, stash_v, is0, is1, os0, os1):
        _topk_body(
            rpw, logits_hbm, out_hbm, (lv0, lv1), (ov0, ov1), stash_v, (is0, is1), (os0, os1)
        )

    return _sc_topk


_NSLICE = 2


def kernel(global_features, W1, b1, W2, b2):
    nrows = _N // _NSLICE
    sc_topk = _make_sc_topk(nrows)
    outs = []
    for s in range(_NSLICE):
        logits = _mlp_logits(global_features, W1, b1, W2, b2, s * nrows, nrows)
        outs.append(sc_topk(logits))
    return jnp.concatenate(outs, axis=0)


# speedup vs baseline: 1.2050x; 1.0059x over previous
import functools

import jax
import jax.numpy as jnp
from jax import lax
from jax.experimental import pallas as pl
from jax.experimental.pallas import tpu as pltpu
from jax.experimental.pallas import tpu_sc as plsc

_N, _D, _H, _E = 32768, 768, 128, 64


_BN = 1024


def _mlp_body(x_ref, w1_ref, b1_ref, w2_ref, b2_ref, out_ref):
    h = jnp.dot(x_ref[...], w1_ref[...], preferred_element_type=jnp.float32)
    h = h + b1_ref[...]
    h = h * jax.nn.sigmoid(h)
    out_ref[...] = (
        jnp.dot(h, w2_ref[...], preferred_element_type=jnp.float32) + b2_ref[...]
    )


def _mlp_logits(x, w1, b1, w2, b2, row0, nrows):
    blk0 = row0 // _BN
    return pl.pallas_call(
        _mlp_body,
        grid=(nrows // _BN,),
        in_specs=[
            pl.BlockSpec((_BN, _D), lambda i: (blk0 + i, 0)),
            pl.BlockSpec((_D, _H), lambda i: (0, 0)),
            pl.BlockSpec((1, _H), lambda i: (0, 0)),
            pl.BlockSpec((_H, _E), lambda i: (0, 0)),
            pl.BlockSpec((1, _E), lambda i: (0, 0)),
        ],
        out_specs=pl.BlockSpec((_BN, _E), lambda i: (i, 0)),
        out_shape=jax.ShapeDtypeStruct((nrows, _E), jnp.float32),
    )(x, w1, b1.reshape(1, _H), w2, b2.reshape(1, _E))


_NC, _NS, _L = 2, 16, 16
_NW = _NC * _NS
_RPW = _N // _NW
_CH = 128
_G = _CH // _L
_NCHUNK = _RPW // _CH
_IL = 4


def _topk_body(rpw, logits_hbm, out_hbm, log_v, out_v, stash_v, in_sem, out_sem):
    wid = lax.axis_index("s") * _NC + lax.axis_index("c")
    lanes = lax.iota(jnp.int32, _L)
    zeros = jnp.zeros((_L,), jnp.float32)
    neg_inf = jnp.full((_L,), -jnp.inf, jnp.float32)
    izero = jnp.zeros((_L,), jnp.int32)
    nchunk = rpw // _CH

    def _roff(ci):
        return wid * rpw + ci * _CH

    def _zero(i, _):
        out_v[0][i // 4, pl.ds((i % 4) * _L, _L)] = zeros
        out_v[1][i // 4, pl.ds((i % 4) * _L, _L)] = zeros
        return 0

    lax.fori_loop(0, (_CH * _E) // _L, _zero, 0)

    def _compute(p, stash):
        @plsc.parallel_loop(0, _G, unroll=_IL)
        def _group(g):
            rl = g * _L + lanes
            m1, m2, i1, i2 = neg_inf, neg_inf, izero, izero
            for e in range(_E):
                ev = jnp.full((_L,), e, jnp.int32)
                v = plsc.load_gather(log_v[p], [rl, ev])
                gt1 = v > m1
                gt2 = v > m2
                m2n = jnp.where(gt1, m1, jnp.where(gt2, v, m2))
                i2n = jnp.where(gt1, i1, jnp.where(gt2, ev, i2))
                m1 = jnp.where(gt1, v, m1)
                i1 = jnp.where(gt1, ev, i1)
                m2, i2 = m2n, i2n
            t = jnp.exp(m2 - m1)
            denom = 1.0 + t
            plsc.store_scatter(out_v[p], [rl, i1], 1.0 / denom)
            plsc.store_scatter(out_v[p], [rl, i2], t / denom)
            if stash:
                stash_v[pl.ds((p * _G + g) * 2 * _L, _L)] = i1
                stash_v[pl.ds((p * _G + g) * 2 * _L + _L, _L)] = i2

    def _unset_zeros(p):
        def _unset(g, _):
            rl = g * _L + lanes
            s0 = pl.ds((p * _G + g) * 2 * _L, _L)
            s1 = pl.ds((p * _G + g) * 2 * _L + _L, _L)
            plsc.store_scatter(out_v[p], [rl, stash_v[s0]], zeros)
            plsc.store_scatter(out_v[p], [rl, stash_v[s1]], zeros)
            return 0

        lax.fori_loop(0, _G, _unset, 0)

    for ci in range(min(2, nchunk)):
        pltpu.async_copy(
            logits_hbm.at[pl.ds(_roff(ci), _CH)], log_v[ci % 2], in_sem[ci % 2]
        )
    for ci in range(nchunk):
        p = ci % 2
        pltpu.make_async_copy(
            logits_hbm.at[pl.ds(_roff(ci), _CH)], log_v[p], in_sem[p]
        ).wait()
        if ci >= 2:
            pltpu.make_async_copy(
                out_v[p], out_hbm.at[pl.ds(_roff(ci - 2), _CH)], out_sem[p]
            ).wait()
            _unset_zeros(p)
        _compute(p, stash=ci + 2 < nchunk)
        pltpu.async_copy(out_v[p], out_hbm.at[pl.ds(_roff(ci), _CH)], out_sem[p])
        if ci + 2 < nchunk:
            pltpu.async_copy(
                logits_hbm.at[pl.ds(_roff(ci + 2), _CH)], log_v[p], in_sem[p]
            )
    for ci in range(max(0, nchunk - 2), nchunk):
        p = ci % 2
        pltpu.make_async_copy(
            out_v[p], out_hbm.at[pl.ds(_roff(ci), _CH)], out_sem[p]
        ).wait()


@functools.lru_cache(maxsize=None)
def _make_sc_topk(nrows):
    rpw = nrows // _NW

    @functools.partial(
        pl.kernel,
        out_type=jax.ShapeDtypeStruct((nrows, _E), jnp.float32),
        mesh=plsc.VectorSubcoreMesh(
            core_axis_name="c", subcore_axis_name="s", num_cores=_NC, num_subcores=_NS
        ),
        scratch_types=[
            pltpu.VMEM((_CH, _E), jnp.float32),
            pltpu.VMEM((_CH, _E), jnp.float32),
            pltpu.VMEM((_CH, _E), jnp.float32),
            pltpu.VMEM((_CH, _E), jnp.float32),
            pltpu.VMEM((2 * _G * 2 * _L,), jnp.int32),
            pltpu.SemaphoreType.DMA,
            pltpu.SemaphoreType.DMA,
            pltpu.SemaphoreType.DMA,
            pltpu.SemaphoreType.DMA,
        ],
        compiler_params=pltpu.CompilerParams(needs_layout_passes=False),
    )
    def _sc_topk(logits_hbm, out_hbm, lv0, lv1, ov0, ov1, stash_v, is0, is1, os0, os1):
        _topk_body(
            rpw, logits_hbm, out_hbm, (lv0, lv1), (ov0, ov1), stash_v, (is0, is1), (os0, os1)
        )

    return _sc_topk


_NSLICE = 1


def kernel(global_features, W1, b1, W2, b2):
    nrows = _N // _NSLICE
    sc_topk = _make_sc_topk(nrows)
    outs = []
    for s in range(_NSLICE):
        logits = _mlp_logits(global_features, W1, b1, W2, b2, s * nrows, nrows)
        outs.append(sc_topk(logits))
    return jnp.concatenate(outs, axis=0)
